# baseline (speedup 1.0000x reference)
import functools

import jax
import jax.numpy as jnp
from jax.experimental import pallas as pl
from jax.experimental.pallas import tpu as pltpu

DIM = 768
HIDDEN = 1536
E = 8
S = 2048
TBLK = 256
NB = S // TBLK


def _dense_body(rw_ref, rb_ref, x_ref, w1_ref, w2_ref, w3_ref, out_ref,
                aux_ref, i1s, i2s, was, wbs):
    e = pl.program_id(0)
    b = pl.program_id(1)
    x = x_ref[...]
    sl = pl.ds(b * TBLK, TBLK)

    @pl.when(e == 0)
    def _route():
        logits = jax.lax.dot_general(
            x, rw_ref[...], (((1,), (1,)), ((), ())),
            preferred_element_type=jnp.float32) + rb_ref[...]
        scores = jax.nn.sigmoid(logits)
        iota = jax.lax.broadcasted_iota(jnp.int32, scores.shape, 1)
        m1 = jnp.max(scores, axis=1, keepdims=True)
        i1 = jnp.min(jnp.where(scores == m1, iota, E), axis=1, keepdims=True)
        masked = jnp.where(iota == i1, -jnp.inf, scores)
        m2 = jnp.max(masked, axis=1, keepdims=True)
        i2 = jnp.min(jnp.where(masked == m2, iota, E), axis=1, keepdims=True)
        denom = m1 + m2 + 1e-6
        i1s[sl, :] = i1
        i2s[sl, :] = i2
        was[sl, :] = m1 / denom
        wbs[sl, :] = m2 / denom
        aux_val = jnp.sum(logits * logits) * (0.01 / (S * E))

        @pl.when(b == 0)
        def _():
            aux_ref[0, 0] = aux_val

        @pl.when(b > 0)
        def _():
            aux_ref[0, 0] += aux_val

    coef = (jnp.where(i1s[sl, :] == e, was[sl, :], 0.0)
            + jnp.where(i2s[sl, :] == e, wbs[sl, :], 0.0))

    xb = x.astype(jnp.bfloat16)
    h1 = jax.lax.dot_general(xb, w1_ref[0].astype(jnp.bfloat16),
                             (((1,), (1,)), ((), ())),
                             preferred_element_type=jnp.float32)
    h2 = jax.lax.dot_general(xb, w2_ref[0].astype(jnp.bfloat16),
                             (((1,), (1,)), ((), ())),
                             preferred_element_type=jnp.float32)
    h = (h1 * jax.nn.sigmoid(h1) * h2).astype(jnp.bfloat16)
    y = jax.lax.dot_general(h, w3_ref[0].astype(jnp.bfloat16),
                            (((1,), (1,)), ((), ())),
                            preferred_element_type=jnp.float32)

    @pl.when(e == 0)
    def _():
        out_ref[sl, :] = y * coef

    @pl.when(e > 0)
    def _():
        out_ref[sl, :] += y * coef


@jax.jit
def _moe(x2d, router_w, router_b2d, w1, w2, w3):
    out, aux = pl.pallas_call(
        _dense_body,
        grid=(E, NB),
        in_specs=[
            pl.BlockSpec((E, DIM), lambda e, b: (0, 0)),
            pl.BlockSpec((1, E), lambda e, b: (0, 0)),
            pl.BlockSpec((TBLK, DIM), lambda e, b: (b, 0)),
            pl.BlockSpec((1, HIDDEN, DIM), lambda e, b: (e, 0, 0)),
            pl.BlockSpec((1, HIDDEN, DIM), lambda e, b: (e, 0, 0)),
            pl.BlockSpec((1, DIM, HIDDEN), lambda e, b: (e, 0, 0)),
        ],
        out_specs=[
            pl.BlockSpec((S, DIM), lambda e, b: (0, 0)),
            pl.BlockSpec(memory_space=pltpu.SMEM, block_shape=(1, 1),
                         index_map=lambda e, b: (0, 0)),
        ],
        out_shape=[
            jax.ShapeDtypeStruct((S, DIM), jnp.float32),
            jax.ShapeDtypeStruct((1, 1), jnp.float32),
        ],
        scratch_shapes=[
            pltpu.VMEM((S, 1), jnp.int32),
            pltpu.VMEM((S, 1), jnp.int32),
            pltpu.VMEM((S, 1), jnp.float32),
            pltpu.VMEM((S, 1), jnp.float32),
        ],
    )(router_w, router_b2d, x2d, w1, w2, w3)
    return out, aux


def kernel(x, router_w, router_b, W12, W3):
    x2d = x.reshape(S, DIM)
    w1 = W12[:, :HIDDEN, :]
    w2 = W12[:, HIDDEN:, :]
    out, aux = _moe(x2d, router_w, router_b.reshape(1, E), w1, w2, W3)
    return out.reshape(1, S, DIM), aux.reshape(())


# speedup vs baseline: 1.1689x; 1.1689x over previous
import functools

import jax
import jax.numpy as jnp
from jax.experimental import pallas as pl
from jax.experimental.pallas import tpu as pltpu

DIM = 768
HIDDEN = 1536
E = 8
S = 2048
TBLK = 512
NB = S // TBLK


def _dense_body(rw_ref, rb_ref, x_ref, w1_ref, w2_ref, w3_ref, out_ref, aux_ref):
    e = pl.program_id(0)
    b = pl.program_id(1)
    x = x_ref[...]

    logits = jax.lax.dot_general(
        x, rw_ref[...], (((1,), (1,)), ((), ())),
        preferred_element_type=jnp.float32) + rb_ref[...]
    scores = jax.nn.sigmoid(logits)
    iota = jax.lax.broadcasted_iota(jnp.int32, scores.shape, 1)
    m1 = jnp.max(scores, axis=1, keepdims=True)
    i1 = jnp.min(jnp.where(scores == m1, iota, E), axis=1, keepdims=True)
    masked = jnp.where(iota == i1, -jnp.inf, scores)
    m2 = jnp.max(masked, axis=1, keepdims=True)
    i2 = jnp.min(jnp.where(masked == m2, iota, E), axis=1, keepdims=True)
    denom = m1 + m2 + 1e-6
    coef = (jnp.where(i1 == e, m1 / denom, 0.0)
            + jnp.where(i2 == e, m2 / denom, 0.0))

    xb = x.astype(jnp.bfloat16)
    h1 = jax.lax.dot_general(xb, w1_ref[0].astype(jnp.bfloat16),
                             (((1,), (1,)), ((), ())),
                             preferred_element_type=jnp.float32)
    h2 = jax.lax.dot_general(xb, w2_ref[0].astype(jnp.bfloat16),
                             (((1,), (1,)), ((), ())),
                             preferred_element_type=jnp.float32)
    h = (h1 * jax.nn.sigmoid(h1) * h2).astype(jnp.bfloat16)
    y = jax.lax.dot_general(h, w3_ref[0].astype(jnp.bfloat16),
                            (((1,), (1,)), ((), ())),
                            preferred_element_type=jnp.float32)

    @pl.when((e == 0) & (b == 0))
    def _init():
        out_ref[...] = jnp.zeros_like(out_ref)
        aux_ref[0, 0] = 0.0

    @pl.when(e == 0)
    def _aux():
        aux_ref[0, 0] += jnp.sum(logits * logits) * (0.01 / (S * E))

    out_ref[pl.ds(b * TBLK, TBLK), :] += y * coef


@jax.jit
def _moe(x2d, router_w, router_b2d, w1, w2, w3):
    out, aux = pl.pallas_call(
        _dense_body,
        grid=(E, NB),
        in_specs=[
            pl.BlockSpec((E, DIM), lambda e, b: (0, 0)),
            pl.BlockSpec((1, E), lambda e, b: (0, 0)),
            pl.BlockSpec((TBLK, DIM), lambda e, b: (b, 0)),
            pl.BlockSpec((1, HIDDEN, DIM), lambda e, b: (e, 0, 0)),
            pl.BlockSpec((1, HIDDEN, DIM), lambda e, b: (e, 0, 0)),
            pl.BlockSpec((1, DIM, HIDDEN), lambda e, b: (e, 0, 0)),
        ],
        out_specs=[
            pl.BlockSpec((S, DIM), lambda e, b: (0, 0)),
            pl.BlockSpec(memory_space=pltpu.SMEM, block_shape=(1, 1),
                         index_map=lambda e, b: (0, 0)),
        ],
        out_shape=[
            jax.ShapeDtypeStruct((S, DIM), jnp.float32),
            jax.ShapeDtypeStruct((1, 1), jnp.float32),
        ],
    )(router_w, router_b2d, x2d, w1, w2, w3)
    return out, aux


def kernel(x, router_w, router_b, W12, W3):
    x2d = x.reshape(S, DIM)
    w1 = W12[:, :HIDDEN, :]
    w2 = W12[:, HIDDEN:, :]
    out, aux = _moe(x2d, router_w, router_b.reshape(1, E), w1, w2, W3)
    return out.reshape(1, S, DIM), aux.reshape(())


# speedup vs baseline: 1.1922x; 1.0200x over previous
import functools

import jax
import jax.numpy as jnp
from jax.experimental import pallas as pl
from jax.experimental.pallas import tpu as pltpu

DIM = 768
HIDDEN = 1536
E = 8
S = 2048
TBLK = 1024
NB = S // TBLK


def _dense_body(rw_ref, rb_ref, x_ref, w1_ref, w2_ref, w3_ref, out_ref, aux_ref):
    e = pl.program_id(0)
    b = pl.program_id(1)
    x = x_ref[...]

    logits = jax.lax.dot_general(
        x, rw_ref[...], (((1,), (1,)), ((), ())),
        preferred_element_type=jnp.float32) + rb_ref[...]
    scores = jax.nn.sigmoid(logits)
    iota = jax.lax.broadcasted_iota(jnp.int32, scores.shape, 1)
    m1 = jnp.max(scores, axis=1, keepdims=True)
    i1 = jnp.min(jnp.where(scores == m1, iota, E), axis=1, keepdims=True)
    masked = jnp.where(iota == i1, -jnp.inf, scores)
    m2 = jnp.max(masked, axis=1, keepdims=True)
    i2 = jnp.min(jnp.where(masked == m2, iota, E), axis=1, keepdims=True)
    denom = m1 + m2 + 1e-6
    coef = (jnp.where(i1 == e, m1 / denom, 0.0)
            + jnp.where(i2 == e, m2 / denom, 0.0))

    xb = x.astype(jnp.bfloat16)
    h1 = jax.lax.dot_general(xb, w1_ref[0].astype(jnp.bfloat16),
                             (((1,), (1,)), ((), ())),
                             preferred_element_type=jnp.float32)
    h2 = jax.lax.dot_general(xb, w2_ref[0].astype(jnp.bfloat16),
                             (((1,), (1,)), ((), ())),
                             preferred_element_type=jnp.float32)
    h = (h1 * jax.nn.sigmoid(h1) * h2).astype(jnp.bfloat16)
    y = jax.lax.dot_general(h, w3_ref[0].astype(jnp.bfloat16),
                            (((1,), (1,)), ((), ())),
                            preferred_element_type=jnp.float32)

    @pl.when((e == 0) & (b == 0))
    def _init():
        out_ref[...] = jnp.zeros_like(out_ref)
        aux_ref[0, 0] = 0.0

    @pl.when(e == 0)
    def _aux():
        aux_ref[0, 0] += jnp.sum(logits * logits) * (0.01 / (S * E))

    out_ref[pl.ds(b * TBLK, TBLK), :] += y * coef


@jax.jit
def _moe(x2d, router_w, router_b2d, w1, w2, w3):
    out, aux = pl.pallas_call(
        _dense_body,
        grid=(E, NB),
        in_specs=[
            pl.BlockSpec((E, DIM), lambda e, b: (0, 0)),
            pl.BlockSpec((1, E), lambda e, b: (0, 0)),
            pl.BlockSpec((TBLK, DIM), lambda e, b: (b, 0)),
            pl.BlockSpec((1, HIDDEN, DIM), lambda e, b: (e, 0, 0)),
            pl.BlockSpec((1, HIDDEN, DIM), lambda e, b: (e, 0, 0)),
            pl.BlockSpec((1, DIM, HIDDEN), lambda e, b: (e, 0, 0)),
        ],
        out_specs=[
            pl.BlockSpec((S, DIM), lambda e, b: (0, 0)),
            pl.BlockSpec(memory_space=pltpu.SMEM, block_shape=(1, 1),
                         index_map=lambda e, b: (0, 0)),
        ],
        out_shape=[
            jax.ShapeDtypeStruct((S, DIM), jnp.float32),
            jax.ShapeDtypeStruct((1, 1), jnp.float32),
        ],
    )(router_w, router_b2d, x2d, w1, w2, w3)
    return out, aux


def kernel(x, router_w, router_b, W12, W3):
    x2d = x.reshape(S, DIM)
    w1 = W12[:, :HIDDEN, :]
    w2 = W12[:, HIDDEN:, :]
    out, aux = _moe(x2d, router_w, router_b.reshape(1, E), w1, w2, W3)
    return out.reshape(1, S, DIM), aux.reshape(())
